# Initial kernel scaffold; baseline (speedup 1.0000x reference)
import jax
import jax.numpy as jnp
from jax.experimental import pallas as pl


def kernel(T, L, D, avg_f, conductivity, time_step, edge_index):
    raise NotImplementedError("write your pallas kernel here")


# speedup vs baseline: 641.1627x; 641.1627x over previous
import functools
import math

import jax
import jax.numpy as jnp
from jax import lax
from jax.experimental import pallas as pl
from jax.experimental.pallas import tpu as pltpu
from jax.experimental.pallas import tpu_sc as plsc

NC = 2
NS = 16
NW = NC * NS

LANES = 16
CHUNK = 4000
STAGE = 6240


def _sc_edge_kernel(N, E, t_hbm, l_hbm, d_hbm, f_hbm, cond_hbm, ts_hbm,
                    src_hbm, dst_hbm, part_hbm,
                    t_sp, w_sp, acc_sp,
                    l_v, d_v, f_v, w_v,
                    src_v, dst_v, cond_v, tsrc_v, tdst_v, wsrc_v, en_v,
                    ts_v, sem_lin, sem_g):
    cid = lax.axis_index("c")
    sid = lax.axis_index("s")
    wid = sid * NC + cid
    per_worker = E // NW
    n_chunks = per_worker // CHUNK

    pltpu.sync_copy(ts_hbm, ts_v.at[pl.ds(0, 1)])
    ts = ts_v[pl.ds(0, LANES)][0]
    pi_half_dt = jnp.float32(math.pi * 0.5) * ts

    lo = sid * STAGE
    cnt = STAGE + jnp.where(sid == NS - 1, N - NS * STAGE, 0)

    d1 = pltpu.async_copy(l_hbm.at[pl.ds(lo, cnt)], l_v.at[pl.ds(0, cnt)], sem_lin)
    d2 = pltpu.async_copy(d_hbm.at[pl.ds(lo, cnt)], d_v.at[pl.ds(0, cnt)], sem_lin)
    d3 = pltpu.async_copy(f_hbm.at[pl.ds(lo, cnt)], f_v.at[pl.ds(0, cnt)], sem_lin)
    d1.wait(); d2.wait(); d3.wait()

    def _stage_body(j, _):
        s = pl.ds(j * LANES, LANES)
        w_v[s] = l_v[s] * d_v[s] * f_v[s] * pi_half_dt
        return 0
    lax.fori_loop(0, cnt // LANES, _stage_body, 0)
    pltpu.sync_copy(w_v.at[pl.ds(0, cnt)], w_sp.at[pl.ds(lo, cnt)])

    pltpu.sync_copy(t_hbm.at[pl.ds(lo, cnt)], d_v.at[pl.ds(0, cnt)])
    pltpu.sync_copy(d_v.at[pl.ds(0, cnt)], t_sp.at[pl.ds(lo, cnt)])

    def _zero_body(j, _):
        l_v[pl.ds(j * LANES, LANES)] = jnp.zeros((LANES,), jnp.float32)
        return 0
    lax.fori_loop(0, cnt // LANES, _zero_body, 0)
    pltpu.sync_copy(l_v.at[pl.ds(0, cnt)], acc_sp.at[pl.ds(lo, cnt)])

    plsc.subcore_barrier()

    base = wid * per_worker

    def _chunk_body(k, _):
        off = base + k * CHUNK
        e1 = pltpu.async_copy(src_hbm.at[pl.ds(off, CHUNK)], src_v, sem_lin)
        e2 = pltpu.async_copy(dst_hbm.at[pl.ds(off, CHUNK)], dst_v, sem_lin)
        e3 = pltpu.async_copy(cond_hbm.at[pl.ds(off, CHUNK)], cond_v, sem_lin)
        e1.wait(); e2.wait(); e3.wait()

        g1 = pltpu.async_copy(t_sp.at[src_v], tsrc_v, sem_g)
        g2 = pltpu.async_copy(t_sp.at[dst_v], tdst_v, sem_g)
        g3 = pltpu.async_copy(w_sp.at[src_v], wsrc_v, sem_g)
        g1.wait(); g2.wait(); g3.wait()

        def _vec_body(j, _):
            s = pl.ds(j * LANES, LANES)
            t_s = tsrc_v[s]
            t_d = tdst_v[s]
            dt = jnp.maximum(t_s - t_d, jnp.float32(0.0))
            en_v[s] = dt * cond_v[s] * wsrc_v[s] * (t_d * t_d * t_d)
            return 0
        lax.fori_loop(0, CHUNK // LANES, _vec_body, 0)

        pltpu.sync_copy(en_v, acc_sp.at[dst_v], add=True)
        return 0

    lax.fori_loop(0, n_chunks, _chunk_body, 0)

    plsc.subcore_barrier()

    pltpu.sync_copy(acc_sp.at[pl.ds(lo, cnt)], l_v.at[pl.ds(0, cnt)])
    pltpu.sync_copy(l_v.at[pl.ds(0, cnt)], part_hbm.at[pl.ds(cid * N + lo, cnt)])


def _combine_kernel(p_ref, o_ref):
    o_ref[...] = p_ref[0, :] + p_ref[1, :]


def kernel(T, L, D, avg_f, conductivity, time_step, edge_index):
    N = T.shape[0]
    E = conductivity.shape[0]
    src = edge_index[0]
    dst = edge_index[1]

    mesh = plsc.VectorSubcoreMesh(core_axis_name="c", subcore_axis_name="s")
    sc_fn = pl.kernel(
        functools.partial(_sc_edge_kernel, N, E),
        out_type=jax.ShapeDtypeStruct((NC * N,), jnp.float32),
        mesh=mesh,
        scratch_types=[
            pltpu.VMEM_SHARED((N,), jnp.float32),
            pltpu.VMEM_SHARED((N,), jnp.float32),
            pltpu.VMEM_SHARED((N,), jnp.float32),
            pltpu.VMEM((STAGE + 160,), jnp.float32),
            pltpu.VMEM((STAGE + 160,), jnp.float32),
            pltpu.VMEM((STAGE + 160,), jnp.float32),
            pltpu.VMEM((STAGE + 160,), jnp.float32),
            pltpu.VMEM((CHUNK,), jnp.int32),
            pltpu.VMEM((CHUNK,), jnp.int32),
            pltpu.VMEM((CHUNK,), jnp.float32),
            pltpu.VMEM((CHUNK,), jnp.float32),
            pltpu.VMEM((CHUNK,), jnp.float32),
            pltpu.VMEM((CHUNK,), jnp.float32),
            pltpu.VMEM((CHUNK,), jnp.float32),
            pltpu.VMEM((LANES,), jnp.float32),
            pltpu.SemaphoreType.DMA,
            pltpu.SemaphoreType.DMA,
        ],
    )
    partials = sc_fn(T, L, D, avg_f, conductivity, time_step, src, dst)

    out = pl.pallas_call(
        _combine_kernel,
        out_shape=jax.ShapeDtypeStruct((N,), jnp.float32),
    )(partials.reshape(NC, N))
    return out
